# Initial kernel scaffold; baseline (speedup 1.0000x reference)
import jax
import jax.numpy as jnp
from jax.experimental import pallas as pl


def kernel(x, edge_index, batch, edge_weight, W1, b1, W2, b2):
    raise NotImplementedError("write your pallas kernel here")


# speedup vs baseline: 59.0550x; 59.0550x over previous
import functools

import jax
import jax.numpy as jnp
from jax import lax
from jax.experimental import pallas as pl
from jax.experimental.pallas import tpu as pltpu
from jax.experimental.pallas import tpu_sc as plsc

N = 10000
NP = 10240
E = 320000
DIM_H = 512
B = 64
NC = 2
NS = 16
NW = NC * NS
F = 8
FH = 4

_EPT_A = E // NW
_CH_A = 2000

_mesh = plsc.VectorSubcoreMesh(core_axis_name="c", subcore_axis_name="s")
_sc_params = pltpu.CompilerParams(needs_layout_passes=False)


@functools.partial(
    pl.kernel,
    out_type=jax.ShapeDtypeStruct((NW, NP), jnp.float32),
    mesh=_mesh,
    compiler_params=_sc_params,
    scratch_types=[
        pltpu.VMEM((NP,), jnp.float32),
        pltpu.VMEM((_CH_A,), jnp.int32),
        pltpu.VMEM((_CH_A,), jnp.float32),
    ],
)
def _deg_kernel(col_hbm, ew_hbm, out_hbm, degbuf, colb, ewb):
    c = lax.axis_index("c")
    s = lax.axis_index("s")
    wid = s * NC + c

    def zero_body(i, _):
        degbuf[pl.ds(i * 16, 16)] = jnp.zeros((16,), jnp.float32)
        return _

    lax.fori_loop(0, NP // 16, zero_body, None)

    def chunk_body(ch, _):
        base = wid * _EPT_A + ch * _CH_A
        pltpu.sync_copy(col_hbm.at[pl.ds(base, _CH_A)], colb)
        pltpu.sync_copy(ew_hbm.at[pl.ds(base, _CH_A)], ewb)

        def grp_body(g, _):
            colv = colb[pl.ds(g * 16, 16)]
            ewv = ewb[pl.ds(g * 16, 16)]
            plsc.addupdate_scatter(degbuf, [colv], ewv)
            return _

        lax.fori_loop(0, _CH_A // 16, grp_body, None)
        return _

    lax.fori_loop(0, _EPT_A // _CH_A, chunk_body, None)
    pltpu.sync_copy(degbuf, out_hbm.at[wid])


_EPT_C = E // NS
_CH_C = 2000


@functools.partial(
    pl.kernel,
    out_type=jax.ShapeDtypeStruct((NC, NS, FH, NP), jnp.float32),
    mesh=_mesh,
    compiler_params=_sc_params,
    scratch_types=[
        pltpu.VMEM((FH, NP), jnp.float32),
        pltpu.VMEM((FH, NP), jnp.float32),
        pltpu.VMEM((_CH_C,), jnp.int32),
        pltpu.VMEM((_CH_C,), jnp.int32),
        pltpu.VMEM((_CH_C,), jnp.float32),
    ],
)
def _agg_kernel(row_hbm, col_hbm, ew_hbm, xp_hbm, out_hbm,
                xph, th, rowb, colb, ewb):
    c = lax.axis_index("c")
    s = lax.axis_index("s")

    pltpu.sync_copy(xp_hbm.at[c], xph)

    for r in range(FH):
        def zero_body(i, _, r=r):
            th[r, pl.ds(i * 16, 16)] = jnp.zeros((16,), jnp.float32)
            return _
        lax.fori_loop(0, NP // 16, zero_body, None)

    iov = lax.iota(jnp.int32, 16)
    io4 = lax.shift_right_logical(iov, 2)
    iom4 = lax.bitwise_and(iov, 3)

    def chunk_body(ch, _):
        base = s * _EPT_C + ch * _CH_C
        pltpu.sync_copy(row_hbm.at[pl.ds(base, _CH_C)], rowb)
        pltpu.sync_copy(col_hbm.at[pl.ds(base, _CH_C)], colb)
        pltpu.sync_copy(ew_hbm.at[pl.ds(base, _CH_C)], ewb)

        def grp_body(g, _):
            eidx = g * 4 + io4
            rowv = plsc.load_gather(rowb, [eidx])
            colv = plsc.load_gather(colb, [eidx])
            ewv = plsc.load_gather(ewb, [eidx])
            xv = plsc.load_gather(xph, [iom4, rowv])
            plsc.addupdate_scatter(th, [iom4, colv], xv * ewv)
            return _

        lax.fori_loop(0, _CH_C // 4, grp_body, None)
        return _

    lax.fori_loop(0, _EPT_C // _CH_C, chunk_body, None)
    pltpu.sync_copy(th, out_hbm.at[c, s])


def _scale_body(degp_ref, x8t_ref, d_ref, xpt_ref):
    deg = jnp.sum(degp_ref[...], axis=0, keepdims=True) + 1.0
    d = jnp.where(deg > 0, lax.rsqrt(jnp.maximum(deg, 1e-30)), 0.0)
    d_ref[...] = d
    xpt_ref[...] = x8t_ref[...] * d


def _scale_call(degp, x8t):
    return pl.pallas_call(
        _scale_body,
        out_shape=(
            jax.ShapeDtypeStruct((1, NP), jnp.float32),
            jax.ShapeDtypeStruct((F, NP), jnp.float32),
        ),
    )(degp, x8t)


_NB = 10
_BN = NP // _NB


def _tail_body(tpart_ref, xpt_ref, d_ref, batch_ref, w1_ref, b1_ref,
               w2_ref, b2_ref, out_ref, acc, cnt):
    i = pl.program_id(0)

    @pl.when(i == 0)
    def _():
        acc[...] = jnp.zeros_like(acc)
        cnt[...] = jnp.zeros_like(cnt)

    tb = jnp.sum(tpart_ref[...], axis=1).reshape(F, _BN)
    sb = d_ref[...] * (tb + xpt_ref[...])
    hb = lax.dot_general(sb, w1_ref[...], (((0,), (0,)), ((), ())),
                         preferred_element_type=jnp.float32)
    hb = jnp.maximum(hb + b1_ref[...], 0.0)
    bids = batch_ref[...]
    gi = lax.broadcasted_iota(jnp.int32, (B, _BN), 0)
    oh = (gi == bids).astype(jnp.float32)
    acc[...] += lax.dot_general(oh, hb, (((1,), (0,)), ((), ())),
                                preferred_element_type=jnp.float32)
    cnt[...] += jnp.sum(oh, axis=1, keepdims=True)

    @pl.when(i == _NB - 1)
    def _():
        hg = acc[...] / jnp.maximum(cnt[...], 1.0)
        o = lax.dot_general(hg, w2_ref[...], (((1,), (0,)), ((), ())),
                            preferred_element_type=jnp.float32) + b2_ref[...]
        m = jnp.max(o, axis=1, keepdims=True)
        lse = m + jnp.log(jnp.sum(jnp.exp(o - m), axis=1, keepdims=True))
        out_ref[...] = o - lse


def _tail_call(tpart, xpt, d, batch2d, w1p, b1, w2, b2):
    return pl.pallas_call(
        _tail_body,
        grid=(_NB,),
        in_specs=[
            pl.BlockSpec((NC, NS, FH, _BN), lambda i: (0, 0, 0, i)),
            pl.BlockSpec((F, _BN), lambda i: (0, i)),
            pl.BlockSpec((1, _BN), lambda i: (0, i)),
            pl.BlockSpec((1, _BN), lambda i: (0, i)),
            pl.BlockSpec((F, DIM_H), lambda i: (0, 0)),
            pl.BlockSpec((1, DIM_H), lambda i: (0, 0)),
            pl.BlockSpec((DIM_H, 2), lambda i: (0, 0)),
            pl.BlockSpec((1, 2), lambda i: (0, 0)),
        ],
        out_specs=pl.BlockSpec((B, 2), lambda i: (0, 0)),
        out_shape=jax.ShapeDtypeStruct((B, 2), jnp.float32),
        scratch_shapes=[
            pltpu.VMEM((B, DIM_H), jnp.float32),
            pltpu.VMEM((B, 1), jnp.float32),
        ],
    )(tpart, xpt, d, batch2d, w1p, b1, w2, b2)


def kernel(x, edge_index, batch, edge_weight, W1, b1, W2, b2):
    ei = edge_index.astype(jnp.int32)
    row = ei[0]
    col = ei[1]
    ew = edge_weight.astype(jnp.float32)
    batch2d = jnp.concatenate(
        [batch.astype(jnp.int32),
         jnp.full((NP - N,), B, jnp.int32)]).reshape(1, NP)

    x8t = jnp.zeros((F, NP), jnp.float32).at[:7, :N].set(
        x.T.astype(jnp.float32))

    degp = _deg_kernel(col, ew)
    d, xpt = _scale_call(degp, x8t)
    xp_halves = xpt.reshape(NC, FH, NP)
    tpart = _agg_kernel(row, col, ew, xp_halves)

    w1p = jnp.concatenate(
        [W1.astype(jnp.float32), jnp.zeros((1, DIM_H), jnp.float32)], axis=0)
    out = _tail_call(tpart, xpt, d, batch2d, w1p,
                     b1.reshape(1, DIM_H), W2, b2.reshape(1, 2))
    return out


# speedup vs baseline: 94.0718x; 1.5930x over previous
import functools

import jax
import jax.numpy as jnp
from jax import lax
from jax.experimental import pallas as pl
from jax.experimental.pallas import tpu as pltpu
from jax.experimental.pallas import tpu_sc as plsc

N = 10000
NP = 10240
E = 320000
DIM_H = 512
B = 64
NC = 2
NS = 16
NW = NC * NS
F = 8
FH = 4

_EPT_A = E // NW
_CH_A = 2000

_mesh = plsc.VectorSubcoreMesh(core_axis_name="c", subcore_axis_name="s")
_sc_params = pltpu.CompilerParams(needs_layout_passes=False)


@functools.partial(
    pl.kernel,
    out_type=jax.ShapeDtypeStruct((NW, NP), jnp.float32),
    mesh=_mesh,
    compiler_params=_sc_params,
    scratch_types=[
        pltpu.VMEM((NP,), jnp.float32),
        pltpu.VMEM((_CH_A,), jnp.int32),
        pltpu.VMEM((_CH_A,), jnp.float32),
    ],
)
def _deg_kernel(col_hbm, ew_hbm, out_hbm, degbuf, colb, ewb):
    c = lax.axis_index("c")
    s = lax.axis_index("s")
    wid = s * NC + c

    @plsc.parallel_loop(0, NP // 16, unroll=8)
    def _(i):
        degbuf[pl.ds(i * 16, 16)] = jnp.zeros((16,), jnp.float32)

    def chunk_body(ch, _):
        base = wid * _EPT_A + ch * _CH_A
        pltpu.sync_copy(col_hbm.at[pl.ds(base, _CH_A)], colb)
        pltpu.sync_copy(ew_hbm.at[pl.ds(base, _CH_A)], ewb)

        @plsc.parallel_loop(0, _CH_A // 16, unroll=8)
        def _(g):
            colv = colb[pl.ds(g * 16, 16)]
            ewv = ewb[pl.ds(g * 16, 16)]
            plsc.addupdate_scatter(degbuf, [colv], ewv)

        return _

    lax.fori_loop(0, _EPT_A // _CH_A, chunk_body, None)
    pltpu.sync_copy(degbuf, out_hbm.at[wid])


_EPT_C = E // NS
_CH_C = 2000


@functools.partial(
    pl.kernel,
    out_type=jax.ShapeDtypeStruct((NC, NS, FH, NP), jnp.float32),
    mesh=_mesh,
    compiler_params=_sc_params,
    scratch_types=[
        pltpu.VMEM((FH, NP), jnp.float32),
        pltpu.VMEM((FH, NP), jnp.float32),
        pltpu.VMEM((_CH_C,), jnp.int32),
        pltpu.VMEM((_CH_C,), jnp.int32),
        pltpu.VMEM((_CH_C,), jnp.float32),
    ],
)
def _agg_kernel(row_hbm, col_hbm, ew_hbm, xp_hbm, out_hbm,
                xph, th, rowb, colb, ewb):
    c = lax.axis_index("c")
    s = lax.axis_index("s")

    pltpu.sync_copy(xp_hbm.at[c], xph)

    for r in range(FH):
        @plsc.parallel_loop(0, NP // 16, unroll=8)
        def _(i, r=r):
            th[r, pl.ds(i * 16, 16)] = jnp.zeros((16,), jnp.float32)

    iov = lax.iota(jnp.int32, 16)
    io4 = lax.shift_right_logical(iov, 2)
    iom4 = lax.bitwise_and(iov, 3)
    exp_idx = [io4 + 4 * q for q in range(4)]

    def chunk_body(ch, _):
        base = s * _EPT_C + ch * _CH_C
        pltpu.sync_copy(row_hbm.at[pl.ds(base, _CH_C)], rowb)
        pltpu.sync_copy(col_hbm.at[pl.ds(base, _CH_C)], colb)
        pltpu.sync_copy(ew_hbm.at[pl.ds(base, _CH_C)], ewb)

        @plsc.parallel_loop(0, _CH_C // 16, unroll=2)
        def _(k):
            rowv = rowb[pl.ds(k * 16, 16)]
            colv = colb[pl.ds(k * 16, 16)]
            ewv = ewb[pl.ds(k * 16, 16)]
            for q in range(4):
                rq = rowv[exp_idx[q]]
                cq = colv[exp_idx[q]]
                eq = ewv[exp_idx[q]]
                xv = plsc.load_gather(xph, [iom4, rq])
                plsc.addupdate_scatter(th, [iom4, cq], xv * eq)

        return _

    lax.fori_loop(0, _EPT_C // _CH_C, chunk_body, None)
    pltpu.sync_copy(th, out_hbm.at[c, s])


def _scale_body(degp_ref, x8t_ref, d_ref, xpt_ref):
    deg = jnp.sum(degp_ref[...], axis=0, keepdims=True) + 1.0
    d = jnp.where(deg > 0, lax.rsqrt(jnp.maximum(deg, 1e-30)), 0.0)
    d_ref[...] = d
    xpt_ref[...] = x8t_ref[...] * d


def _scale_call(degp, x8t):
    return pl.pallas_call(
        _scale_body,
        out_shape=(
            jax.ShapeDtypeStruct((1, NP), jnp.float32),
            jax.ShapeDtypeStruct((F, NP), jnp.float32),
        ),
    )(degp, x8t)


_NB = 10
_BN = NP // _NB


def _tail_body(tpart_ref, xpt_ref, d_ref, batch_ref, w1_ref, b1_ref,
               w2_ref, b2_ref, out_ref, acc, cnt):
    i = pl.program_id(0)

    @pl.when(i == 0)
    def _():
        acc[...] = jnp.zeros_like(acc)
        cnt[...] = jnp.zeros_like(cnt)

    tb = jnp.sum(tpart_ref[...], axis=1).reshape(F, _BN)
    sb = d_ref[...] * (tb + xpt_ref[...])
    hb = lax.dot_general(sb, w1_ref[...], (((0,), (0,)), ((), ())),
                         preferred_element_type=jnp.float32)
    hb = jnp.maximum(hb + b1_ref[...], 0.0)
    bids = batch_ref[...]
    gi = lax.broadcasted_iota(jnp.int32, (B, _BN), 0)
    oh = (gi == bids).astype(jnp.float32)
    acc[...] += lax.dot_general(oh, hb, (((1,), (0,)), ((), ())),
                                preferred_element_type=jnp.float32)
    cnt[...] += jnp.sum(oh, axis=1, keepdims=True)

    @pl.when(i == _NB - 1)
    def _():
        hg = acc[...] / jnp.maximum(cnt[...], 1.0)
        o = lax.dot_general(hg, w2_ref[...], (((1,), (0,)), ((), ())),
                            preferred_element_type=jnp.float32) + b2_ref[...]
        m = jnp.max(o, axis=1, keepdims=True)
        lse = m + jnp.log(jnp.sum(jnp.exp(o - m), axis=1, keepdims=True))
        out_ref[...] = o - lse


def _tail_call(tpart, xpt, d, batch2d, w1p, b1, w2, b2):
    return pl.pallas_call(
        _tail_body,
        grid=(_NB,),
        in_specs=[
            pl.BlockSpec((NC, NS, FH, _BN), lambda i: (0, 0, 0, i)),
            pl.BlockSpec((F, _BN), lambda i: (0, i)),
            pl.BlockSpec((1, _BN), lambda i: (0, i)),
            pl.BlockSpec((1, _BN), lambda i: (0, i)),
            pl.BlockSpec((F, DIM_H), lambda i: (0, 0)),
            pl.BlockSpec((1, DIM_H), lambda i: (0, 0)),
            pl.BlockSpec((DIM_H, 2), lambda i: (0, 0)),
            pl.BlockSpec((1, 2), lambda i: (0, 0)),
        ],
        out_specs=pl.BlockSpec((B, 2), lambda i: (0, 0)),
        out_shape=jax.ShapeDtypeStruct((B, 2), jnp.float32),
        scratch_shapes=[
            pltpu.VMEM((B, DIM_H), jnp.float32),
            pltpu.VMEM((B, 1), jnp.float32),
        ],
    )(tpart, xpt, d, batch2d, w1p, b1, w2, b2)


def kernel(x, edge_index, batch, edge_weight, W1, b1, W2, b2):
    ei = edge_index.astype(jnp.int32)
    row = ei[0]
    col = ei[1]
    ew = edge_weight.astype(jnp.float32)
    batch2d = jnp.concatenate(
        [batch.astype(jnp.int32),
         jnp.full((NP - N,), B, jnp.int32)]).reshape(1, NP)

    x8t = jnp.zeros((F, NP), jnp.float32).at[:7, :N].set(
        x.T.astype(jnp.float32))

    degp = _deg_kernel(col, ew)
    d, xpt = _scale_call(degp, x8t)
    xp_halves = xpt.reshape(NC, FH, NP)
    tpart = _agg_kernel(row, col, ew, xp_halves)

    w1p = jnp.concatenate(
        [W1.astype(jnp.float32), jnp.zeros((1, DIM_H), jnp.float32)], axis=0)
    out = _tail_call(tpart, xpt, d, batch2d, w1p,
                     b1.reshape(1, DIM_H), W2, b2.reshape(1, 2))
    return out
